# Initial kernel scaffold; baseline (speedup 1.0000x reference)
import jax
import jax.numpy as jnp
from jax.experimental import pallas as pl


def kernel(z_dyn, z_static, dt, ut, W1, b1, W2, b2, W3, b3, Wa, ba, Wb, bb, Wc, bc, Wd, bd):
    raise NotImplementedError("write your pallas kernel here")


# speedup vs baseline: 1.7498x; 1.7498x over previous
import functools

import jax
import jax.numpy as jnp
from jax.experimental import pallas as pl
from jax.experimental.pallas import tpu as pltpu

_DYN = 96
_STAT = 32
_U = 8
_NOBS = 13
_TOTAL_IN = _DYN + _STAT + 1
_HZ = 128
_H1 = 200
_H2 = 200
_LANE = 128

_TB = 256


def _fused_body(x_ref, v1_ref,
                w1_ref, b1_ref, w2_ref, b2_ref, w3_ref, b3_ref,
                wab_ref, bab_ref, wcd_ref, bcd_ref,
                z_ref, y_ref):
    x = x_ref[...]
    h = jnp.maximum(
        jnp.dot(x, w1_ref[...], preferred_element_type=jnp.float32)
        + b1_ref[...], 0.0)
    h = jnp.maximum(
        jnp.dot(h, w2_ref[...], preferred_element_type=jnp.float32)
        + b2_ref[...], 0.0)
    hz = (jnp.dot(h, w3_ref[...], preferred_element_type=jnp.float32)
          + b3_ref[...])

    ab = (jnp.dot(hz, wab_ref[...], preferred_element_type=jnp.float32)
          + bab_ref[...])
    ab3 = ab.reshape(_TB, _DYN, _LANE)
    v1 = v1_ref[...]
    z_next = jnp.sum(ab3 * v1[:, None, :], axis=2)

    cd = (jnp.dot(hz, wcd_ref[...], preferred_element_type=jnp.float32)
          + bcd_ref[...])
    cd3 = cd.reshape(_TB, _NOBS, _LANE)
    v2 = jnp.concatenate([z_next, v1[:, _DYN:]], axis=1)
    yt = jnp.sum(cd3 * v2[:, None, :], axis=2)

    z_ref[...] = z_next
    y_ref[...] = yt


@jax.jit
def kernel(z_dyn, z_static, dt, ut, W1, b1, W2, b2, W3, b3,
           Wa, ba, Wb, bb, Wc, bc, Wd, bd):
    B = z_dyn.shape[0]
    f32 = jnp.float32

    x = jnp.concatenate([z_dyn, z_static, dt], axis=-1)
    pad_b = jnp.zeros((B, _LANE - _DYN - _U), dtype=f32)
    v1 = jnp.concatenate([z_dyn, ut * dt, pad_b], axis=-1)

    def interleave(Wx, bx, Wy, by, rows):
        padw = jnp.zeros((_HZ, rows, _LANE - _DYN - _U), dtype=f32)
        W = jnp.concatenate(
            [Wx.reshape(_HZ, rows, _DYN), Wy.reshape(_HZ, rows, _U), padw],
            axis=2).reshape(_HZ, rows * _LANE)
        padb = jnp.zeros((rows, _LANE - _DYN - _U), dtype=f32)
        bvec = jnp.concatenate(
            [bx.reshape(rows, _DYN), by.reshape(rows, _U), padb],
            axis=1).reshape(1, rows * _LANE)
        return W, bvec

    Wab, bab = interleave(Wa, ba, Wb, bb, _DYN)
    Wcd, bcd = interleave(Wc, bc, Wd, bd, _NOBS)

    grid = (B // _TB,)
    row_spec = lambda n: pl.BlockSpec((_TB, n), lambda i: (i, 0))
    w_spec = lambda shp: pl.BlockSpec(shp, lambda i: (0, 0))

    z_next, yt = pl.pallas_call(
        _fused_body,
        grid=grid,
        in_specs=[
            row_spec(_TOTAL_IN),
            row_spec(_LANE),
            w_spec((_TOTAL_IN, _H1)), w_spec((1, _H1)),
            w_spec((_H1, _H2)), w_spec((1, _H2)),
            w_spec((_H2, _HZ)), w_spec((1, _HZ)),
            w_spec((_HZ, _DYN * _LANE)), w_spec((1, _DYN * _LANE)),
            w_spec((_HZ, _NOBS * _LANE)), w_spec((1, _NOBS * _LANE)),
        ],
        out_specs=[row_spec(_DYN), row_spec(_NOBS)],
        out_shape=[
            jax.ShapeDtypeStruct((B, _DYN), f32),
            jax.ShapeDtypeStruct((B, _NOBS), f32),
        ],
        compiler_params=pltpu.CompilerParams(
            dimension_semantics=("arbitrary",)),
    )(x, v1, W1, b1.reshape(1, -1), W2, b2.reshape(1, -1),
      W3, b3.reshape(1, -1), Wab, bab, Wcd, bcd)

    return (z_next, yt)
